# baseline (speedup 1.0000x reference)
import functools

import jax
import jax.numpy as jnp
from jax import lax
from jax.experimental import pallas as pl
from jax.experimental.pallas import tpu as pltpu
from jax.experimental.pallas import tpu_sc as plsc

B = 16384
C = 50
E = 64
SIGMA = 0.01

_NC = 2
_NS = 16
_NW = _NC * _NS
_BPW = B // _NW
_KCH = _BPW // 128

_sc_mesh = plsc.VectorSubcoreMesh(core_axis_name="c", subcore_axis_name="s")


@functools.partial(
    pl.kernel,
    mesh=_sc_mesh,
    compiler_params=pltpu.CompilerParams(use_tc_tiling_on_sc=False),
    out_type=jax.ShapeDtypeStruct((_NW, _KCH, 128, E), jnp.float32),
    scratch_types=[
        pltpu.VMEM((_KCH, 128), jnp.int32),
        pltpu.VMEM((_KCH, 128, E), jnp.float32),
        pltpu.SemaphoreType.DMA,
    ],
)
def _sc_gather(subject_hbm, table_hbm, out_hbm, idx_v, rows_v, sem):
    wid = lax.axis_index("s") * _NC + lax.axis_index("c")
    pltpu.sync_copy(subject_hbm.at[wid], idx_v)
    copies = [
        pltpu.async_copy(table_hbm.at[idx_v.at[j]], rows_v.at[j], sem)
        for j in range(_KCH)
    ]
    for c in copies:
        c.wait()
    pltpu.sync_copy(rows_v, out_hbm.at[wid])


def _concat_body(emb_ref, noise_ref, x_ref, out_ref):
    out_ref[:, 0, :] = emb_ref[...] + noise_ref[...]
    out_ref[:, 1:, :] = x_ref[...]


_BB = 512


def _tc_concat(emb, noise, x):
    return pl.pallas_call(
        _concat_body,
        grid=(B // _BB,),
        in_specs=[
            pl.BlockSpec((_BB, E), lambda i: (i, 0)),
            pl.BlockSpec((_BB, E), lambda i: (i, 0)),
            pl.BlockSpec((_BB, C, E), lambda i: (i, 0, 0)),
        ],
        out_specs=pl.BlockSpec((_BB, C + 1, E), lambda i: (i, 0, 0)),
        out_shape=jax.ShapeDtypeStruct((B, C + 1, E), jnp.float32),
    )(emb, noise, x)


def kernel(x, subject, table):
    noise = jax.random.normal(jax.random.key(42), (B, 1, E), dtype=jnp.float32)
    noise = (noise * SIGMA).reshape(B, E)
    subject_r = subject.astype(jnp.int32).reshape(_NW, _KCH, 128)
    emb = _sc_gather(subject_r, table).reshape(B, E)
    return _tc_concat(emb, noise, x)


# speedup vs baseline: 1.2007x; 1.2007x over previous
import functools

import jax
import jax.numpy as jnp
from jax import lax
from jax.experimental import pallas as pl
from jax.experimental.pallas import tpu as pltpu
from jax.experimental.pallas import tpu_sc as plsc

B = 16384
C = 50
E = 64
SIGMA = 0.01

_NC = 2
_NS = 16
_NW = _NC * _NS
_BPW = B // _NW

_sc_mesh = plsc.VectorSubcoreMesh(core_axis_name="c", subcore_axis_name="s")


@functools.partial(
    pl.kernel,
    mesh=_sc_mesh,
    out_type=jax.ShapeDtypeStruct((_NW, _BPW, E), jnp.float32),
    scratch_types=[
        pltpu.VMEM((_BPW,), jnp.int32),
        pltpu.VMEM((_BPW, E), jnp.float32),
        pltpu.SemaphoreType.DMA,
    ],
)
def _sc_gather(subject_hbm, table_hbm, out_hbm, idx_v, rows_v, sem):
    wid = lax.axis_index("s") * _NC + lax.axis_index("c")
    pltpu.sync_copy(subject_hbm.at[wid], idx_v)

    def body(j, _):
        base = j * 16
        idx16 = idx_v[pl.ds(base, 16)]
        for k in range(16):
            s = idx16[k]
            pltpu.async_copy(
                table_hbm.at[pl.ds(s, 1)], rows_v.at[pl.ds(base + k, 1)], sem
            )
        return 0

    lax.fori_loop(0, _BPW // 16, body, 0)
    pltpu.make_async_copy(table_hbm.at[pl.ds(0, _BPW)], rows_v, sem).wait()
    pltpu.sync_copy(rows_v, out_hbm.at[wid])


def _concat_body(emb_ref, noise_ref, x_ref, out_ref):
    out_ref[:, 0, :] = emb_ref[...] + noise_ref[...]
    out_ref[:, 1:, :] = x_ref[...]


_BB = 128


def _tc_concat(emb, noise, x):
    return pl.pallas_call(
        _concat_body,
        grid=(B // _BB,),
        in_specs=[
            pl.BlockSpec((_BB, E), lambda i: (i, 0)),
            pl.BlockSpec((_BB, E), lambda i: (i, 0)),
            pl.BlockSpec((_BB, C, E), lambda i: (i, 0, 0)),
        ],
        out_specs=pl.BlockSpec((_BB, C + 1, E), lambda i: (i, 0, 0)),
        out_shape=jax.ShapeDtypeStruct((B, C + 1, E), jnp.float32),
    )(emb, noise, x)


def kernel(x, subject, table):
    noise = jax.random.normal(jax.random.key(42), (B, 1, E), dtype=jnp.float32)
    noise = (noise * SIGMA).reshape(B, E)
    subject_r = subject.astype(jnp.int32).reshape(_NW, _BPW)
    emb = _sc_gather(subject_r, table).reshape(B, E)
    return _tc_concat(emb, noise, x)
